# baseline (speedup 1.0000x reference)
import jax
import jax.numpy as jnp
from jax.experimental import pallas as pl
from jax.experimental.pallas import tpu as pltpu

_NCH = 16


def _body(idx_ref, arr_any, elem_ref, out_any, sems, psem):
    M = arr_any.shape[0]
    ch = M // _NCH
    idx = idx_ref[0]
    owner = idx // ch
    descs = []
    for k in range(_NCH):
        d = pltpu.make_async_copy(
            arr_any.at[pl.ds(k * ch, ch)], out_any.at[pl.ds(k * ch, ch)],
            sems.at[k])
        d.start()
        descs.append(d)
    for k in range(_NCH):
        descs[k].wait()

        @pl.when(owner == k)
        def _(k=k):
            pltpu.make_async_copy(elem_ref, out_any.at[pl.ds(idx, 1)],
                                  psem).start()
    pltpu.make_async_copy(elem_ref, out_any.at[pl.ds(idx, 1)], psem).wait()


def kernel(arr, index, element):
    M, N = arr.shape
    idx = jnp.asarray(index, jnp.int32).reshape((1,))
    elem2d = element.reshape((1, N))
    return pl.pallas_call(
        _body,
        grid_spec=pltpu.PrefetchScalarGridSpec(
            num_scalar_prefetch=1,
            grid=(1,),
            in_specs=[
                pl.BlockSpec(memory_space=pl.ANY),
                pl.BlockSpec((1, N), lambda i, idx_ref: (0, 0)),
            ],
            out_specs=pl.BlockSpec(memory_space=pl.ANY),
            scratch_shapes=[
                pltpu.SemaphoreType.DMA((_NCH,)),
                pltpu.SemaphoreType.DMA,
            ],
        ),
        out_shape=jax.ShapeDtypeStruct((M, N), arr.dtype),
    )(idx, arr, elem2d)


# speedup vs baseline: 28.3708x; 28.3708x over previous
import jax
import jax.numpy as jnp
from jax.experimental import pallas as pl
from jax.experimental.pallas import tpu as pltpu

_BLK = 256


def _body(idx_ref, elem_ref, arr_ref, out_ref):
    i = pl.program_id(0)
    local = idx_ref[0] - i * _BLK
    rows = jax.lax.broadcasted_iota(jnp.int32, (_BLK, 1), 0)
    out_ref[...] = jnp.where(rows == local, elem_ref[...], arr_ref[...])


def kernel(arr, index, element):
    M, N = arr.shape
    idx = jnp.asarray(index, jnp.int32).reshape((1,))
    elem2d = element.reshape((1, N))
    return pl.pallas_call(
        _body,
        grid_spec=pltpu.PrefetchScalarGridSpec(
            num_scalar_prefetch=1,
            grid=(M // _BLK,),
            in_specs=[
                pl.BlockSpec((1, N), lambda i, idx_ref: (0, 0)),
                pl.BlockSpec((_BLK, N), lambda i, idx_ref: (i, 0)),
            ],
            out_specs=pl.BlockSpec((_BLK, N), lambda i, idx_ref: (i, 0)),
        ),
        out_shape=jax.ShapeDtypeStruct((M, N), arr.dtype),
    )(idx, elem2d, arr)


# speedup vs baseline: 39.9679x; 1.4088x over previous
import jax
import jax.numpy as jnp
from jax.experimental import pallas as pl
from jax.experimental.pallas import tpu as pltpu

_BLK = 1024


def _body(idx_ref, elem_ref, arr_ref, out_ref):
    i = pl.program_id(0)
    local = idx_ref[0] - i * _BLK
    rows = jax.lax.broadcasted_iota(jnp.int32, (_BLK, 1), 0)
    out_ref[...] = jnp.where(rows == local, elem_ref[...], arr_ref[...])


def kernel(arr, index, element):
    M, N = arr.shape
    idx = jnp.asarray(index, jnp.int32).reshape((1,))
    elem2d = element.reshape((1, N))
    return pl.pallas_call(
        _body,
        grid_spec=pltpu.PrefetchScalarGridSpec(
            num_scalar_prefetch=1,
            grid=(M // _BLK,),
            in_specs=[
                pl.BlockSpec((1, N), lambda i, idx_ref: (0, 0)),
                pl.BlockSpec((_BLK, N), lambda i, idx_ref: (i, 0)),
            ],
            out_specs=pl.BlockSpec((_BLK, N), lambda i, idx_ref: (i, 0)),
        ),
        out_shape=jax.ShapeDtypeStruct((M, N), arr.dtype),
    )(idx, elem2d, arr)


# speedup vs baseline: 44.3956x; 1.1108x over previous
import jax
import jax.numpy as jnp
from jax.experimental import pallas as pl
from jax.experimental.pallas import tpu as pltpu

_BLK = 2048


def _body(idx_ref, elem_ref, arr_ref, out_ref):
    i = pl.program_id(0)
    local = idx_ref[0] - i * _BLK
    rows = jax.lax.broadcasted_iota(jnp.int32, (_BLK, 1), 0)
    out_ref[...] = jnp.where(rows == local, elem_ref[...], arr_ref[...])


def kernel(arr, index, element):
    M, N = arr.shape
    idx = jnp.asarray(index, jnp.int32).reshape((1,))
    elem2d = element.reshape((1, N))
    return pl.pallas_call(
        _body,
        grid_spec=pltpu.PrefetchScalarGridSpec(
            num_scalar_prefetch=1,
            grid=(M // _BLK,),
            in_specs=[
                pl.BlockSpec((1, N), lambda i, idx_ref: (0, 0)),
                pl.BlockSpec((_BLK, N), lambda i, idx_ref: (i, 0)),
            ],
            out_specs=pl.BlockSpec((_BLK, N), lambda i, idx_ref: (i, 0)),
        ),
        out_shape=jax.ShapeDtypeStruct((M, N), arr.dtype),
    )(idx, elem2d, arr)
